# Initial kernel scaffold; baseline (speedup 1.0000x reference)
import jax
import jax.numpy as jnp
from jax.experimental import pallas as pl


def kernel(x, edge_index, Wl0, bl0, Wr0, Wl1, bl1, Wr1):
    raise NotImplementedError("write your pallas kernel here")


# speedup vs baseline: 2.7893x; 2.7893x over previous
import functools

import jax
import jax.numpy as jnp
from jax import lax
from jax.experimental import pallas as pl
from jax.experimental.pallas import tpu as pltpu
from jax.experimental.pallas import tpu_sc as plsc

NC = 2
NS = 16
NW = NC * NS
C = 128


def _sc_segment_sums(x_p, srcs, dsts, zrows, zrow1, ones_c, *, n_pad, h, k,
                     with_cnt):
    rpt = n_pad // NS
    mesh = plsc.VectorSubcoreMesh(core_axis_name="c", subcore_axis_name="s")

    out_type = [jax.ShapeDtypeStruct((NC * n_pad, h), jnp.float32)]
    scratch = [
        pltpu.VMEM((k, C), jnp.int32),
        pltpu.VMEM((k, C), jnp.int32),
        pltpu.VMEM((C, h), jnp.float32),
        pltpu.VMEM_SHARED((n_pad, h), jnp.float32),
    ]
    if with_cnt:
        out_type.append(jax.ShapeDtypeStruct((NC * n_pad,), jnp.float32))
        scratch.append(pltpu.VMEM((C,), jnp.float32))
        scratch.append(pltpu.VMEM_SHARED((n_pad,), jnp.float32))

    def body_cnt(x_hbm, src_hbm, dst_hbm, zr_hbm, z1_hbm, on_hbm,
                 sums_hbm, cnt_hbm, srcv, dstv, buf, acc, onesv, cacc):
        cid = lax.axis_index("c")
        sid = lax.axis_index("s")
        wid = cid * NS + sid
        pltpu.sync_copy(zr_hbm, acc.at[pl.ds(sid * rpt, rpt)])
        pltpu.sync_copy(z1_hbm, cacc.at[pl.ds(sid * rpt, rpt)])
        pltpu.sync_copy(on_hbm, onesv)
        pltpu.sync_copy(src_hbm.at[pl.ds(wid * k, k)], srcv)
        pltpu.sync_copy(dst_hbm.at[pl.ds(wid * k, k)], dstv)
        plsc.subcore_barrier()

        def step(j, _):
            pltpu.sync_copy(x_hbm.at[srcv.at[j]], buf)
            pltpu.sync_copy(buf, acc.at[dstv.at[j]], add=True)
            pltpu.sync_copy(onesv, cacc.at[dstv.at[j]], add=True)
            return _

        lax.fori_loop(0, k, step, None)
        plsc.subcore_barrier()
        base = cid * n_pad + sid * rpt
        pltpu.sync_copy(acc.at[pl.ds(sid * rpt, rpt)],
                        sums_hbm.at[pl.ds(base, rpt)])
        pltpu.sync_copy(cacc.at[pl.ds(sid * rpt, rpt)],
                        cnt_hbm.at[pl.ds(base, rpt)])

    def body_nocnt(x_hbm, src_hbm, dst_hbm, zr_hbm,
                   sums_hbm, srcv, dstv, buf, acc):
        cid = lax.axis_index("c")
        sid = lax.axis_index("s")
        wid = cid * NS + sid
        pltpu.sync_copy(zr_hbm, acc.at[pl.ds(sid * rpt, rpt)])
        pltpu.sync_copy(src_hbm.at[pl.ds(wid * k, k)], srcv)
        pltpu.sync_copy(dst_hbm.at[pl.ds(wid * k, k)], dstv)
        plsc.subcore_barrier()

        def step(j, _):
            pltpu.sync_copy(x_hbm.at[srcv.at[j]], buf)
            pltpu.sync_copy(buf, acc.at[dstv.at[j]], add=True)
            return _

        lax.fori_loop(0, k, step, None)
        plsc.subcore_barrier()
        base = cid * n_pad + sid * rpt
        pltpu.sync_copy(acc.at[pl.ds(sid * rpt, rpt)],
                        sums_hbm.at[pl.ds(base, rpt)])

    if with_cnt:
        run = pl.kernel(body_cnt, out_type=out_type, mesh=mesh,
                        scratch_types=scratch)
        return run(x_p, srcs, dsts, zrows, zrow1, ones_c)
    run = pl.kernel(body_nocnt, out_type=out_type, mesh=mesh,
                    scratch_types=scratch)
    return run(x_p, srcs, dsts, zrows)


def _prep_rcp(cnt3, *, n_pad):
    def body(c_ref, o_ref):
        i = pl.program_id(0)
        c = c_ref[0, i] + c_ref[1, i]
        r = (1.0 / jnp.maximum(c, 1.0)).reshape(1, 128)
        eye = (lax.broadcasted_iota(jnp.int32, (128, 128), 0)
               == lax.broadcasted_iota(jnp.int32, (128, 128), 1)
               ).astype(jnp.float32)
        col = lax.dot_general(eye, r, (((1,), (1,)), ((), ())),
                              preferred_element_type=jnp.float32)
        o_ref[...] = jnp.broadcast_to(col, (128, 128))

    return pl.pallas_call(
        body,
        grid=(n_pad // 128,),
        in_specs=[pl.BlockSpec((NC, n_pad // 128, 128), lambda i: (0, 0, 0))],
        out_specs=pl.BlockSpec((128, 128), lambda i: (i, 0)),
        out_shape=jax.ShapeDtypeStruct((n_pad, 128), jnp.float32),
    )(cnt3)


def _combine(sums2, rcpb, xin, wl, bl, wr, *, relu, n_pad, h, blk=1024):
    def body(s_ref, r_ref, x_ref, wl_ref, bl_ref, wr_ref, o_ref):
        s = s_ref[0] + s_ref[1]
        agg = s * r_ref[...]
        o = (jnp.dot(agg, wl_ref[...], preferred_element_type=jnp.float32)
             + bl_ref[...]
             + jnp.dot(x_ref[...], wr_ref[...],
                       preferred_element_type=jnp.float32))
        ss = jnp.sum(o * o, axis=-1, keepdims=True)
        inv = 1.0 / jnp.maximum(jnp.sqrt(ss), 1e-12)
        o = o * inv
        if relu:
            o = jnp.maximum(o, 0.0)
        o_ref[...] = o

    d = xin.shape[1]
    return pl.pallas_call(
        body,
        grid=(n_pad // blk,),
        in_specs=[
            pl.BlockSpec((NC, blk, d), lambda i: (0, i, 0)),
            pl.BlockSpec((blk, h), lambda i: (i, 0)),
            pl.BlockSpec((blk, d), lambda i: (i, 0)),
            pl.BlockSpec((d, h), lambda i: (0, 0)),
            pl.BlockSpec((1, h), lambda i: (0, 0)),
            pl.BlockSpec((d, h), lambda i: (0, 0)),
        ],
        out_specs=pl.BlockSpec((blk, h), lambda i: (i, 0)),
        out_shape=jax.ShapeDtypeStruct((n_pad, h), jnp.float32),
    )(sums2, rcpb, xin, wl, bl.reshape(1, h), wr)


def kernel(x, edge_index, Wl0, bl0, Wr0, Wl1, bl1, Wr1):
    n, d = x.shape
    h = Wl0.shape[1]
    e = edge_index.shape[1]

    n_pad = -(-n // (NS * C)) * (NS * C)
    k = -(-(-(-e // (NW * C))) // 8) * 8
    e_pad = NW * C * k
    rpt = n_pad // NS

    src = jnp.concatenate(
        [edge_index[0], jnp.zeros((e_pad - e,), jnp.int32)]).reshape(NW * k, C)
    dst = jnp.concatenate(
        [edge_index[1], jnp.full((e_pad - e,), n, jnp.int32)]).reshape(NW * k, C)
    x_p = jnp.pad(x, ((0, n_pad - n), (0, 0)))
    zrows = jnp.zeros((rpt, d), jnp.float32)
    zrow1 = jnp.zeros((rpt,), jnp.float32)
    ones_c = jnp.ones((C,), jnp.float32)

    sums0, cnt = _sc_segment_sums(x_p, src, dst, zrows, zrow1, ones_c,
                                  n_pad=n_pad, h=d, k=k, with_cnt=True)
    rcpb = _prep_rcp(cnt.reshape(NC, n_pad // 128, 128), n_pad=n_pad)
    h1 = _combine(sums0.reshape(NC, n_pad, d), rcpb, x_p, Wl0, bl0, Wr0,
                  relu=True, n_pad=n_pad, h=h)
    (sums1,) = _sc_segment_sums(h1, src, dst, zrows, zrow1, ones_c,
                                n_pad=n_pad, h=h, k=k, with_cnt=False)
    out = _combine(sums1.reshape(NC, n_pad, h), rcpb, h1, Wl1, bl1, Wr1,
                   relu=False, n_pad=n_pad, h=h)
    return out[:n]


# speedup vs baseline: 3.5083x; 1.2578x over previous
import functools

import jax
import jax.numpy as jnp
from jax import lax
from jax.experimental import pallas as pl
from jax.experimental.pallas import tpu as pltpu
from jax.experimental.pallas import tpu_sc as plsc

NC = 2
NS = 16
NW = NC * NS
C = 128


def _sc_segment_sums(x_p, sd, zrows, zrow1, ones_c, *, n_pad, h, k, with_cnt):
    rpt = n_pad // NS
    mesh = plsc.VectorSubcoreMesh(core_axis_name="c", subcore_axis_name="s")
    assert k % 4 == 0 and k >= 8

    out_type = [jax.ShapeDtypeStruct((NC * n_pad, h), jnp.float32)]
    scratch = [
        pltpu.VMEM((4, 2, C), jnp.int32),
        pltpu.SemaphoreType.DMA((4,)),
        pltpu.VMEM((2, C, h), jnp.float32),
        pltpu.SemaphoreType.DMA((2,)),
        pltpu.VMEM_SHARED((n_pad, h), jnp.float32),
    ]
    if with_cnt:
        out_type.append(jax.ShapeDtypeStruct((NC * n_pad,), jnp.float32))
        scratch.append(pltpu.VMEM((C,), jnp.float32))
        scratch.append(pltpu.VMEM_SHARED((n_pad,), jnp.float32))

    def make_body(with_cnt):
        def body(x_hbm, sd_hbm, zr_hbm, z1_hbm, on_hbm, *refs):
            if with_cnt:
                (sums_hbm, cnt_hbm, idxb, isem, bufs, gsem, acc, onesv,
                 cacc) = refs
            else:
                sums_hbm, idxb, isem, bufs, gsem, acc = refs
            cid = lax.axis_index("c")
            sid = lax.axis_index("s")
            tb = (cid * NS + sid) * k
            pltpu.sync_copy(zr_hbm, acc.at[pl.ds(sid * rpt, rpt)])
            if with_cnt:
                pltpu.sync_copy(z1_hbm, cacc.at[pl.ds(sid * rpt, rpt)])
                pltpu.sync_copy(on_hbm, onesv)
            plsc.subcore_barrier()

            for b in range(4):
                pltpu.async_copy(sd_hbm.at[tb + b], idxb.at[b], isem.at[b])
            for b in range(2):
                pltpu.make_async_copy(sd_hbm.at[tb + b], idxb.at[b],
                                      isem.at[b]).wait()
                pltpu.async_copy(x_hbm.at[idxb.at[b, 0]], bufs.at[b],
                                 gsem.at[b])

            @pl.loop(0, k, step=4)
            def _(j0):
                for b in range(4):
                    j = j0 + b
                    bb = b % 2
                    pltpu.make_async_copy(x_hbm.at[idxb.at[b, 0]],
                                          bufs.at[bb], gsem.at[bb]).wait()
                    pltpu.sync_copy(bufs.at[bb], acc.at[idxb.at[b, 1]],
                                    add=True)
                    if with_cnt:
                        pltpu.sync_copy(onesv, cacc.at[idxb.at[b, 1]],
                                        add=True)

                    @pl.when(j + 4 < k)
                    def _():
                        pltpu.async_copy(sd_hbm.at[tb + j + 4], idxb.at[b],
                                         isem.at[b])

                    @pl.when(j + 2 < k)
                    def _():
                        b2 = (b + 2) % 4
                        pltpu.make_async_copy(sd_hbm.at[tb + j + 2],
                                              idxb.at[b2], isem.at[b2]).wait()
                        pltpu.async_copy(x_hbm.at[idxb.at[b2, 0]],
                                         bufs.at[bb], gsem.at[bb])

            plsc.subcore_barrier()
            base = cid * n_pad + sid * rpt
            pltpu.sync_copy(acc.at[pl.ds(sid * rpt, rpt)],
                            sums_hbm.at[pl.ds(base, rpt)])
            if with_cnt:
                pltpu.sync_copy(cacc.at[pl.ds(sid * rpt, rpt)],
                                cnt_hbm.at[pl.ds(base, rpt)])
        return body

    run = pl.kernel(make_body(with_cnt), out_type=out_type, mesh=mesh,
                    scratch_types=scratch)
    return run(x_p, sd, zrows, zrow1, ones_c)


def _prep_rcp(cnt3, *, n_pad):
    def body(c_ref, o_ref):
        i = pl.program_id(0)
        c = c_ref[0, i] + c_ref[1, i]
        r = (1.0 / jnp.maximum(c, 1.0)).reshape(1, 128)
        eye = (lax.broadcasted_iota(jnp.int32, (128, 128), 0)
               == lax.broadcasted_iota(jnp.int32, (128, 128), 1)
               ).astype(jnp.float32)
        col = lax.dot_general(eye, r, (((1,), (1,)), ((), ())),
                              preferred_element_type=jnp.float32)
        o_ref[...] = jnp.broadcast_to(col, (128, 128))

    return pl.pallas_call(
        body,
        grid=(n_pad // 128,),
        in_specs=[pl.BlockSpec((NC, n_pad // 128, 128), lambda i: (0, 0, 0))],
        out_specs=pl.BlockSpec((128, 128), lambda i: (i, 0)),
        out_shape=jax.ShapeDtypeStruct((n_pad, 128), jnp.float32),
    )(cnt3)


def _combine(sums2, rcpb, xin, wl, bl, wr, *, relu, n_pad, h, blk=1024):
    def body(s_ref, r_ref, x_ref, wl_ref, bl_ref, wr_ref, o_ref):
        s = s_ref[0] + s_ref[1]
        agg = s * r_ref[...]
        o = (jnp.dot(agg, wl_ref[...], preferred_element_type=jnp.float32)
             + bl_ref[...]
             + jnp.dot(x_ref[...], wr_ref[...],
                       preferred_element_type=jnp.float32))
        ss = jnp.sum(o * o, axis=-1, keepdims=True)
        inv = 1.0 / jnp.maximum(jnp.sqrt(ss), 1e-12)
        o = o * inv
        if relu:
            o = jnp.maximum(o, 0.0)
        o_ref[...] = o

    d = xin.shape[1]
    return pl.pallas_call(
        body,
        grid=(n_pad // blk,),
        in_specs=[
            pl.BlockSpec((NC, blk, d), lambda i: (0, i, 0)),
            pl.BlockSpec((blk, h), lambda i: (i, 0)),
            pl.BlockSpec((blk, d), lambda i: (i, 0)),
            pl.BlockSpec((d, h), lambda i: (0, 0)),
            pl.BlockSpec((1, h), lambda i: (0, 0)),
            pl.BlockSpec((d, h), lambda i: (0, 0)),
        ],
        out_specs=pl.BlockSpec((blk, h), lambda i: (i, 0)),
        out_shape=jax.ShapeDtypeStruct((n_pad, h), jnp.float32),
    )(sums2, rcpb, xin, wl, bl.reshape(1, h), wr)


def kernel(x, edge_index, Wl0, bl0, Wr0, Wl1, bl1, Wr1):
    n, d = x.shape
    h = Wl0.shape[1]
    e = edge_index.shape[1]

    n_pad = -(-n // (NS * C)) * (NS * C)
    k = -(-(-(-e // (NW * C))) // 8) * 8
    e_pad = NW * C * k
    rpt = n_pad // NS

    src = jnp.concatenate(
        [edge_index[0], jnp.zeros((e_pad - e,), jnp.int32)]).reshape(NW * k, C)
    dst = jnp.concatenate(
        [edge_index[1], jnp.full((e_pad - e,), n, jnp.int32)]).reshape(NW * k, C)
    sd = jnp.stack([src, dst], axis=1)
    x_p = jnp.pad(x, ((0, n_pad - n), (0, 0)))
    zrows = jnp.zeros((rpt, d), jnp.float32)
    zrow1 = jnp.zeros((rpt,), jnp.float32)
    ones_c = jnp.ones((C,), jnp.float32)

    sums0, cnt = _sc_segment_sums(x_p, sd, zrows, zrow1, ones_c,
                                  n_pad=n_pad, h=d, k=k, with_cnt=True)
    rcpb = _prep_rcp(cnt.reshape(NC, n_pad // 128, 128), n_pad=n_pad)
    h1 = _combine(sums0.reshape(NC, n_pad, d), rcpb, x_p, Wl0, bl0, Wr0,
                  relu=True, n_pad=n_pad, h=h)
    (sums1,) = _sc_segment_sums(h1, sd, zrows, zrow1, ones_c,
                                n_pad=n_pad, h=h, k=k, with_cnt=False)
    out = _combine(sums1.reshape(NC, n_pad, h), rcpb, h1, Wl1, bl1, Wr1,
                   relu=False, n_pad=n_pad, h=h)
    return out[:n]


# speedup vs baseline: 4.0208x; 1.1461x over previous
import functools

import jax
import jax.numpy as jnp
from jax import lax
from jax.experimental import pallas as pl
from jax.experimental.pallas import tpu as pltpu
from jax.experimental.pallas import tpu_sc as plsc

NC = 2
NS = 16
NW = NC * NS
C = 128


def _sc_segment_sums(x_p, sd, zrows, *, n_pad, h, k):
    rpt = n_pad // NS
    mesh = plsc.VectorSubcoreMesh(core_axis_name="c", subcore_axis_name="s")
    assert k % 4 == 0 and k >= 8

    out_type = jax.ShapeDtypeStruct((NC * n_pad, h), jnp.float32)
    scratch = [
        pltpu.VMEM((3, 2, C), jnp.int32),
        pltpu.SemaphoreType.DMA((3,)),
        pltpu.VMEM((3, C, h), jnp.float32),
        pltpu.SemaphoreType.DMA((3,)),
        pltpu.SemaphoreType.DMA((3,)),
        pltpu.VMEM_SHARED((n_pad, h), jnp.float32),
    ]

    def body(x_hbm, sd_hbm, zr_hbm, sums_hbm, idxb, isem, bufs, gsem, ssem,
             acc):
        cid = lax.axis_index("c")
        sid = lax.axis_index("s")
        tb = (cid * NS + sid) * k
        pltpu.sync_copy(zr_hbm, acc.at[pl.ds(sid * rpt, rpt)])
        plsc.subcore_barrier()

        for b in range(2):
            pltpu.async_copy(sd_hbm.at[tb + b], idxb.at[b], isem.at[b])
        for b in range(2):
            pltpu.make_async_copy(sd_hbm.at[tb + b], idxb.at[b],
                                  isem.at[b]).wait()
            pltpu.async_copy(x_hbm.at[idxb.at[b, 0]], bufs.at[b], gsem.at[b])

        @pl.loop(0, k)
        def _(j):
            b = lax.rem(j, 3)
            b2 = lax.rem(j + 2, 3)

            @pl.when(j >= 1)
            def _():
                pltpu.make_async_copy(bufs.at[b2], acc.at[idxb.at[b2, 1]],
                                      ssem.at[b2]).wait()

            @pl.when(j + 2 < k)
            def _():
                pltpu.async_copy(sd_hbm.at[tb + j + 2], idxb.at[b2],
                                 isem.at[b2])

            pltpu.make_async_copy(x_hbm.at[idxb.at[b, 0]], bufs.at[b],
                                  gsem.at[b]).wait()
            pltpu.async_copy(bufs.at[b], acc.at[idxb.at[b, 1]], ssem.at[b],
                             add=True)

            @pl.when(j + 2 < k)
            def _():
                pltpu.make_async_copy(sd_hbm.at[tb + j + 2], idxb.at[b2],
                                      isem.at[b2]).wait()
                pltpu.async_copy(x_hbm.at[idxb.at[b2, 0]], bufs.at[b2],
                                 gsem.at[b2])

        bl_ = (k - 1) % 3
        pltpu.make_async_copy(bufs.at[bl_], acc.at[idxb.at[bl_, 1]],
                              ssem.at[bl_]).wait()
        plsc.subcore_barrier()
        base = cid * n_pad + sid * rpt
        pltpu.sync_copy(acc.at[pl.ds(sid * rpt, rpt)],
                        sums_hbm.at[pl.ds(base, rpt)])

    run = pl.kernel(body, out_type=out_type, mesh=mesh, scratch_types=scratch)
    return run(x_p, sd, zrows)


def _sc_counts(sd, zrow1, ones_c, *, n_pad, k):
    rpt = n_pad // NS
    mesh = plsc.VectorSubcoreMesh(core_axis_name="c", subcore_axis_name="s")

    out_type = jax.ShapeDtypeStruct((NC * n_pad,), jnp.float32)
    scratch = [
        pltpu.VMEM((8, 2, C), jnp.int32),
        pltpu.SemaphoreType.DMA((8,)),
        pltpu.SemaphoreType.DMA((4,)),
        pltpu.VMEM((C,), jnp.float32),
        pltpu.VMEM_SHARED((n_pad,), jnp.float32),
    ]

    def body(sd_hbm, z1_hbm, on_hbm, cnt_hbm, idxb, isem, csem, onesv, cacc):
        cid = lax.axis_index("c")
        sid = lax.axis_index("s")
        tb = (cid * NS + sid) * k
        pltpu.sync_copy(z1_hbm, cacc.at[pl.ds(sid * rpt, rpt)])
        pltpu.sync_copy(on_hbm, onesv)
        plsc.subcore_barrier()

        for b in range(4):
            pltpu.async_copy(sd_hbm.at[tb + b], idxb.at[b], isem.at[b])

        @pl.loop(0, k)
        def _(j):
            b4 = lax.rem(j, 4)
            s8 = lax.rem(j, 8)

            @pl.when(j >= 4)
            def _():
                pltpu.make_async_copy(onesv, cacc.at[idxb.at[s8, 1]],
                                      csem.at[b4]).wait()

            @pl.when(j + 4 < k)
            def _():
                s8n = lax.rem(j + 4, 8)
                pltpu.async_copy(sd_hbm.at[tb + j + 4], idxb.at[s8n],
                                 isem.at[s8n])

            pltpu.make_async_copy(sd_hbm.at[tb + j], idxb.at[s8],
                                  isem.at[s8]).wait()
            pltpu.async_copy(onesv, cacc.at[idxb.at[s8, 1]], csem.at[b4],
                             add=True)

        for b in range(4):
            pltpu.make_async_copy(onesv, cacc.at[idxb.at[b, 1]],
                                  csem.at[b]).wait()
        plsc.subcore_barrier()
        base = cid * n_pad + sid * rpt
        pltpu.sync_copy(cacc.at[pl.ds(sid * rpt, rpt)],
                        cnt_hbm.at[pl.ds(base, rpt)])

    run = pl.kernel(body, out_type=out_type, mesh=mesh, scratch_types=scratch)
    return run(sd, zrow1, ones_c)


def _prep_rcp(cnt3, *, n_pad):
    def body(c_ref, o_ref):
        i = pl.program_id(0)
        c = c_ref[0, i] + c_ref[1, i]
        r = (1.0 / jnp.maximum(c, 1.0)).reshape(1, 128)
        eye = (lax.broadcasted_iota(jnp.int32, (128, 128), 0)
               == lax.broadcasted_iota(jnp.int32, (128, 128), 1)
               ).astype(jnp.float32)
        col = lax.dot_general(eye, r, (((1,), (1,)), ((), ())),
                              preferred_element_type=jnp.float32)
        o_ref[...] = jnp.broadcast_to(col, (128, 128))

    return pl.pallas_call(
        body,
        grid=(n_pad // 128,),
        in_specs=[pl.BlockSpec((NC, n_pad // 128, 128), lambda i: (0, 0, 0))],
        out_specs=pl.BlockSpec((128, 128), lambda i: (i, 0)),
        out_shape=jax.ShapeDtypeStruct((n_pad, 128), jnp.float32),
    )(cnt3)


def _combine(sums2, rcpb, xin, wl, bl, wr, *, relu, n_pad, h, blk):
    def body(s_ref, r_ref, x_ref, wl_ref, bl_ref, wr_ref, o_ref):
        s = s_ref[0] + s_ref[1]
        agg = s * r_ref[...]
        o = (jnp.dot(agg, wl_ref[...], preferred_element_type=jnp.float32)
             + bl_ref[...]
             + jnp.dot(x_ref[...], wr_ref[...],
                       preferred_element_type=jnp.float32))
        ss = jnp.sum(o * o, axis=-1, keepdims=True)
        inv = 1.0 / jnp.maximum(jnp.sqrt(ss), 1e-12)
        o = o * inv
        if relu:
            o = jnp.maximum(o, 0.0)
        o_ref[...] = o

    d = xin.shape[1]
    return pl.pallas_call(
        body,
        grid=(n_pad // blk,),
        in_specs=[
            pl.BlockSpec((NC, blk, d), lambda i: (0, i, 0)),
            pl.BlockSpec((blk, h), lambda i: (i, 0)),
            pl.BlockSpec((blk, d), lambda i: (i, 0)),
            pl.BlockSpec((d, h), lambda i: (0, 0)),
            pl.BlockSpec((1, h), lambda i: (0, 0)),
            pl.BlockSpec((d, h), lambda i: (0, 0)),
        ],
        out_specs=pl.BlockSpec((blk, h), lambda i: (i, 0)),
        out_shape=jax.ShapeDtypeStruct((n_pad, h), jnp.float32),
    )(sums2, rcpb, xin, wl, bl.reshape(1, h), wr)


def kernel(x, edge_index, Wl0, bl0, Wr0, Wl1, bl1, Wr1):
    n, d = x.shape
    h = Wl0.shape[1]
    e = edge_index.shape[1]

    n_pad = -(-(n + 1) // 128) * 128
    blk = n_pad // NS
    k = -(-(-(-e // (NW * C))) // 8) * 8
    e_pad = NW * C * k
    rpt = n_pad // NS

    src = jnp.concatenate(
        [edge_index[0], jnp.zeros((e_pad - e,), jnp.int32)]).reshape(NW * k, C)
    dst = jnp.concatenate(
        [edge_index[1], jnp.full((e_pad - e,), n, jnp.int32)]).reshape(NW * k, C)
    sd = jnp.stack([src, dst], axis=1)
    x_p = jnp.pad(x, ((0, n_pad - n), (0, 0)))
    zrows = jnp.zeros((rpt, d), jnp.float32)
    ones_c = jnp.ones((C,), jnp.float32)

    n_pad_c = -(-(n + 1) // 2048) * 2048
    zrow1 = jnp.zeros((n_pad_c // NS,), jnp.float32)
    cnt = _sc_counts(sd, zrow1, ones_c, n_pad=n_pad_c, k=k)
    sums0 = _sc_segment_sums(x_p, sd, zrows, n_pad=n_pad, h=d, k=k)
    cnt3 = cnt.reshape(NC, n_pad_c)[:, :n_pad].reshape(NC, n_pad // 128, 128)
    rcpb = _prep_rcp(cnt3, n_pad=n_pad)
    h1 = _combine(sums0.reshape(NC, n_pad, d), rcpb, x_p, Wl0, bl0, Wr0,
                  relu=True, n_pad=n_pad, h=h, blk=blk)
    sums1 = _sc_segment_sums(h1, sd, zrows, n_pad=n_pad, h=h, k=k)
    out = _combine(sums1.reshape(NC, n_pad, h), rcpb, h1, Wl1, bl1, Wr1,
                   relu=False, n_pad=n_pad, h=h, blk=blk)
    return out[:n]
